# baseline (speedup 1.0000x reference)
import functools

import jax
import jax.numpy as jnp
from jax import lax
from jax.experimental import pallas as pl
from jax.experimental.pallas import tpu as pltpu
from jax.experimental.pallas import tpu_sc as plsc

_N = 10000
_E = 320000
_F = 128
_H = 64
_G = 128
_C = 10

_NC = 2
_NS = 16
_NW = _NC * _NS
_B = 128
_EPW = _E // _NW
_K = 80
_NP = 10112
_RPT = _NP // _NS

_F32 = jnp.float32
_HI = lax.Precision.HIGHEST
_TC_PARAMS = pltpu.CompilerParams(vmem_limit_bytes=100 * 1024 * 1024)


def _edge_agg(m, zeros, src3, dst3):
    mesh = plsc.VectorSubcoreMesh(core_axis_name="c", subcore_axis_name="s")

    @functools.partial(
        pl.kernel,
        out_type=jax.ShapeDtypeStruct((_NC, _NP, _H), _F32),
        mesh=mesh,
        scratch_types=[
            pltpu.VMEM((_K, _B), jnp.int32),
            pltpu.VMEM((_K, _B), jnp.int32),
            pltpu.VMEM((_B, _H), _F32),
            pltpu.VMEM_SHARED((_NP, _H), _F32),
            pltpu.VMEM_SHARED((_NP, _H), _F32),
            pltpu.SemaphoreType.DMA,
        ],
        compiler_params=pltpu.CompilerParams(use_tc_tiling_on_sc=False),
    )
    def k(m_hbm, z_hbm, src_hbm, dst_hbm, out_hbm, src_v, dst_v, rows_v,
          m_sh, agg_sh, sem):
        c = lax.axis_index("c")
        s = lax.axis_index("s")
        wid = s * _NC + c
        pltpu.sync_copy(z_hbm.at[pl.ds(s * _RPT, _RPT)],
                        agg_sh.at[pl.ds(s * _RPT, _RPT)])
        pltpu.sync_copy(m_hbm.at[pl.ds(s * _RPT, _RPT)],
                        m_sh.at[pl.ds(s * _RPT, _RPT)])
        pltpu.sync_copy(src_hbm.at[wid], src_v)
        pltpu.sync_copy(dst_hbm.at[wid], dst_v)
        plsc.subcore_barrier()

        @pl.loop(0, _K)
        def _(j):
            pltpu.async_copy(m_sh.at[src_v.at[j]], rows_v, sem).wait()
            pltpu.sync_copy(rows_v, agg_sh.at[dst_v.at[j]], add=True)

        plsc.subcore_barrier()
        pltpu.sync_copy(agg_sh.at[pl.ds(s * _RPT, _RPT)],
                        out_hbm.at[c, pl.ds(s * _RPT, _RPT)])

    return k(m, zeros, src3, dst3)


_BF = jnp.bfloat16


def _split2(v):
    hi = v.astype(_BF)
    return hi, (v - hi.astype(_F32)).astype(_BF)


def _odot(ot_bf, v):
    vh, vl = _split2(v)
    return (jnp.dot(ot_bf, vh, preferred_element_type=_F32) +
            jnp.dot(ot_bf, vl, preferred_element_type=_F32))


def _ogather(ot_bf, per_g):
    gh, gl = _split2(per_g)
    dn = (((0,), (0,)), ((), ()))
    return (lax.dot_general(ot_bf, gh, dn, preferred_element_type=_F32) +
            lax.dot_general(ot_bf, gl, dn, preferred_element_type=_F32))


def _dot3(a, b):
    ah, al = _split2(a)
    bh, bl = _split2(b)
    return (jnp.dot(ah, bh, preferred_element_type=_F32) +
            (jnp.dot(ah, bl, preferred_element_type=_F32) +
             jnp.dot(al, bh, preferred_element_type=_F32)))


def _onehot(br):
    n = br.shape[1]
    return (lax.broadcasted_iota(jnp.int32, (_G, n), 0) == br).astype(_BF)


def _norm_stats(xv, ot_bf, ms):
    cnt = jnp.maximum(
        jnp.sum(ot_bf.astype(_F32), axis=1, keepdims=True), 1.0)
    mean = _odot(ot_bf, xv) / cnt
    ex2 = _odot(ot_bf, xv * xv) / cnt
    var = ex2 - mean * mean * (ms * (2.0 - ms))
    return cnt, mean, jnp.sqrt(var + 1e-5)


def _tc_first(x, br, gw, gb, gms, Wr, b1, Ws):
    def body(x_r, br_r, gw_r, gb_r, gms_r, wr_r, b1_r, ws_r, m_o, h_o):
        ot = _onehot(br_r[...])
        _, mean, std = _norm_stats(x_r[...], ot, gms_r[...])
        meanb = _ogather(ot, mean)
        stdb = _ogather(ot, std)
        h = gw_r[...] * (x_r[...] - meanb * gms_r[...]) / stdb + gb_r[...]
        m_o[0:_N, :] = _dot3(h, wr_r[...])
        h_o[...] = h

    return pl.pallas_call(
        body,
        out_shape=[jax.ShapeDtypeStruct((_NP, _H), _F32),
                   jax.ShapeDtypeStruct((_N, _F), _F32)],
        compiler_params=_TC_PARAMS,
    )(x, br, gw, gb, gms, Wr, b1, Ws)


def _tc_mid(agg, s_in, br, gw, gb, gms, Wr, bn, Ws):
    def body(a_r, s_r, br_r, gw_r, gb_r, gms_r, wr_r, bn_r, ws_r, m_o, h_o):
        h = jnp.maximum(a_r[0, :_N, :] + a_r[1, :_N, :] + s_r[...], 0.0)
        ot = _onehot(br_r[...])
        _, mean, std = _norm_stats(h, ot, gms_r[...])
        meanb = _ogather(ot, mean)
        stdb = _ogather(ot, std)
        hn = gw_r[...] * (h - meanb * gms_r[...]) / stdb + gb_r[...]
        m_o[0:_N, :] = _dot3(hn, wr_r[...])
        h_o[...] = hn

    return pl.pallas_call(
        body,
        out_shape=[jax.ShapeDtypeStruct((_NP, _H), _F32),
                   jax.ShapeDtypeStruct((_N, _H), _F32)],
        compiler_params=_TC_PARAMS,
    )(agg, s_in, br, gw, gb, gms, Wr, bn, Ws)


def _tc_s(h, Ws, bn):
    def body(h_r, ws_r, bn_r, s_o):
        s_o[...] = _dot3(h_r[...], ws_r[...]) + bn_r[...]

    return pl.pallas_call(
        body,
        out_shape=jax.ShapeDtypeStruct((_N, _H), _F32),
        compiler_params=_TC_PARAMS,
    )(h, Ws, bn)


def _tc_final(agg, s_in, br, Wd, bd, Wo, bo):
    def body(a_r, s_r, br_r, wd_r, bd_r, wo_r, bo_r, out_o):
        h = jnp.maximum(a_r[0, :_N, :] + a_r[1, :_N, :] + s_r[...], 0.0)
        ot = _onehot(br_r[...])
        cnt = jnp.maximum(
            jnp.sum(ot.astype(_F32), axis=1, keepdims=True), 1.0)
        g = _odot(ot, h) / cnt
        g = jnp.maximum(jnp.dot(g, wd_r[...], precision=_HI) + bd_r[...], 0.0)
        logits = jnp.dot(g, wo_r[...], precision=_HI) + bo_r[...]
        zmax = jnp.max(logits, axis=1, keepdims=True)
        ez = jnp.exp(logits - zmax)
        out_o[...] = ez / jnp.sum(ez, axis=1, keepdims=True)

    return pl.pallas_call(
        body,
        out_shape=jax.ShapeDtypeStruct((_G, _C), _F32),
        compiler_params=_TC_PARAMS,
    )(agg, s_in, br, Wd, bd, Wo, bo)


def kernel(x, edge_index, batch, gn0_w, gn0_b, gn0_ms, W1r, b1, W1s,
           gn1_w, gn1_b, gn1_ms, W2r, b2, W2s, gn2_w, gn2_b, gn2_ms,
           W3r, b3, W3s, Wd, bd, Wo, bo):
    br = batch.reshape(1, _N)
    pad = _NW * _K * _B - _E
    src = jnp.concatenate([edge_index[0], jnp.zeros((pad,), jnp.int32)])
    dst = jnp.concatenate([edge_index[1], jnp.full((pad,), _N, jnp.int32)])
    src3 = src.reshape(_NW, _K, _B)
    dst3 = dst.reshape(_NW, _K, _B)
    zeros = jnp.zeros((_NP, _H), _F32)

    def row(v):
        return v.reshape(1, -1)

    m1, h0 = _tc_first(x, br, row(gn0_w), row(gn0_b), row(gn0_ms),
                       W1r, row(b1), W1s)
    agg1 = _edge_agg(m1, zeros, src3, dst3)
    s1 = _tc_s(h0, W1s, row(b1))
    m2, h1 = _tc_mid(agg1, s1, br, row(gn1_w), row(gn1_b), row(gn1_ms),
                     W2r, row(b2), W2s)
    agg2 = _edge_agg(m2, zeros, src3, dst3)
    s2 = _tc_s(h1, W2s, row(b2))
    m3, h2 = _tc_mid(agg2, s2, br, row(gn2_w), row(gn2_b), row(gn2_ms),
                     W3r, row(b3), W3s)
    agg3 = _edge_agg(m3, zeros, src3, dst3)
    s3 = _tc_s(h2, W3s, row(b3))
    return _tc_final(agg3, s3, br, Wd, bd, Wo, bo)


# speedup vs baseline: 1.0125x; 1.0125x over previous
import functools

import jax
import jax.numpy as jnp
from jax import lax
from jax.experimental import pallas as pl
from jax.experimental.pallas import tpu as pltpu
from jax.experimental.pallas import tpu_sc as plsc

_N = 10000
_E = 320000
_F = 128
_H = 64
_G = 128
_C = 10

_NC = 2
_NS = 16
_NW = _NC * _NS
_B = 128
_EPW = _E // _NW
_K = 80
_NP = 10112
_RPT = _NP // _NS

_F32 = jnp.float32
_HI = lax.Precision.HIGHEST
_TC_PARAMS = pltpu.CompilerParams(vmem_limit_bytes=100 * 1024 * 1024)


def _edge_agg(m, zeros, src3, dst3):
    mesh = plsc.VectorSubcoreMesh(core_axis_name="c", subcore_axis_name="s")

    @functools.partial(
        pl.kernel,
        out_type=jax.ShapeDtypeStruct((_NC, _NP, _H), _F32),
        mesh=mesh,
        scratch_types=[
            pltpu.VMEM((_K, _B), jnp.int32),
            pltpu.VMEM((_K, _B), jnp.int32),
            pltpu.VMEM((_B, _H), _F32),
            pltpu.VMEM_SHARED((_NP, _H), _F32),
            pltpu.VMEM_SHARED((_NP, _H), _F32),
            pltpu.SemaphoreType.DMA,
        ],
        compiler_params=pltpu.CompilerParams(use_tc_tiling_on_sc=False),
    )
    def k(m_hbm, z_hbm, src_hbm, dst_hbm, out_hbm, src_v, dst_v, rows_v,
          m_sh, agg_sh, sem):
        c = lax.axis_index("c")
        s = lax.axis_index("s")
        wid = s * _NC + c
        pltpu.sync_copy(z_hbm.at[pl.ds(s * _RPT, _RPT)],
                        agg_sh.at[pl.ds(s * _RPT, _RPT)])
        pltpu.sync_copy(m_hbm.at[pl.ds(s * _RPT, _RPT)],
                        m_sh.at[pl.ds(s * _RPT, _RPT)])
        pltpu.sync_copy(src_hbm.at[wid], src_v)
        pltpu.sync_copy(dst_hbm.at[wid], dst_v)
        plsc.subcore_barrier()

        @pl.loop(0, _K)
        def _(j):
            pltpu.async_copy(m_sh.at[src_v.at[j]], rows_v, sem).wait()
            pltpu.sync_copy(rows_v, agg_sh.at[dst_v.at[j]], add=True)

        plsc.subcore_barrier()
        pltpu.sync_copy(agg_sh.at[pl.ds(s * _RPT, _RPT)],
                        out_hbm.at[c, pl.ds(s * _RPT, _RPT)])

    return k(m, zeros, src3, dst3)


_BF = jnp.bfloat16


def _split2(v):
    hi = v.astype(_BF)
    return hi, (v - hi.astype(_F32)).astype(_BF)


def _odot(ot_bf, v):
    vh, vl = _split2(v)
    return (jnp.dot(ot_bf, vh, preferred_element_type=_F32) +
            jnp.dot(ot_bf, vl, preferred_element_type=_F32))


def _ogather(ot_bf, per_g):
    gh, gl = _split2(per_g)
    dn = (((0,), (0,)), ((), ()))
    return (lax.dot_general(ot_bf, gh, dn, preferred_element_type=_F32) +
            lax.dot_general(ot_bf, gl, dn, preferred_element_type=_F32))


def _dot3(a, b):
    ah, al = _split2(a)
    bh, bl = _split2(b)
    return (jnp.dot(ah, bh, preferred_element_type=_F32) +
            (jnp.dot(ah, bl, preferred_element_type=_F32) +
             jnp.dot(al, bh, preferred_element_type=_F32)))


def _onehot(br):
    n = br.shape[1]
    return (lax.broadcasted_iota(jnp.int32, (_G, n), 0) == br).astype(_BF)


def _norm_stats(xv, ot_bf, ms):
    cnt = jnp.maximum(
        jnp.sum(ot_bf.astype(_F32), axis=1, keepdims=True), 1.0)
    mean = _odot(ot_bf, xv) / cnt
    ex2 = _odot(ot_bf, xv * xv) / cnt
    var = ex2 - mean * mean * (ms * (2.0 - ms))
    return cnt, mean, jnp.sqrt(var + 1e-5)


def _tc_first(x, br, gw, gb, gms, Wr, b1, Ws):
    def body(x_r, br_r, gw_r, gb_r, gms_r, wr_r, b1_r, ws_r, m_o, s_o):
        ot = _onehot(br_r[...])
        _, mean, std = _norm_stats(x_r[...], ot, gms_r[...])
        meanb = _ogather(ot, mean)
        stdb = _ogather(ot, std)
        h = gw_r[...] * (x_r[...] - meanb * gms_r[...]) / stdb + gb_r[...]
        m_o[0:_N, :] = _dot3(h, wr_r[...])
        s_o[...] = _dot3(h, ws_r[...]) + b1_r[...]

    return pl.pallas_call(
        body,
        out_shape=[jax.ShapeDtypeStruct((_NP, _H), _F32),
                   jax.ShapeDtypeStruct((_N, _H), _F32)],
        compiler_params=_TC_PARAMS,
    )(x, br, gw, gb, gms, Wr, b1, Ws)


def _tc_mid(agg, s_in, br, gw, gb, gms, Wr, bn, Ws):
    def body(a_r, s_r, br_r, gw_r, gb_r, gms_r, wr_r, bn_r, ws_r, m_o, s_o):
        h = jnp.maximum(a_r[0, :_N, :] + a_r[1, :_N, :] + s_r[...], 0.0)
        ot = _onehot(br_r[...])
        _, mean, std = _norm_stats(h, ot, gms_r[...])
        meanb = _ogather(ot, mean)
        stdb = _ogather(ot, std)
        hn = gw_r[...] * (h - meanb * gms_r[...]) / stdb + gb_r[...]
        m_o[0:_N, :] = _dot3(hn, wr_r[...])
        s_o[...] = _dot3(hn, ws_r[...]) + bn_r[...]

    return pl.pallas_call(
        body,
        out_shape=[jax.ShapeDtypeStruct((_NP, _H), _F32),
                   jax.ShapeDtypeStruct((_N, _H), _F32)],
        compiler_params=_TC_PARAMS,
    )(agg, s_in, br, gw, gb, gms, Wr, bn, Ws)


def _tc_final(agg, s_in, br, Wd, bd, Wo, bo):
    def body(a_r, s_r, br_r, wd_r, bd_r, wo_r, bo_r, out_o):
        h = jnp.maximum(a_r[0, :_N, :] + a_r[1, :_N, :] + s_r[...], 0.0)
        ot = _onehot(br_r[...])
        cnt = jnp.maximum(
            jnp.sum(ot.astype(_F32), axis=1, keepdims=True), 1.0)
        g = _odot(ot, h) / cnt
        g = jnp.maximum(jnp.dot(g, wd_r[...], precision=_HI) + bd_r[...], 0.0)
        logits = jnp.dot(g, wo_r[...], precision=_HI) + bo_r[...]
        zmax = jnp.max(logits, axis=1, keepdims=True)
        ez = jnp.exp(logits - zmax)
        out_o[...] = ez / jnp.sum(ez, axis=1, keepdims=True)

    return pl.pallas_call(
        body,
        out_shape=jax.ShapeDtypeStruct((_G, _C), _F32),
        compiler_params=_TC_PARAMS,
    )(agg, s_in, br, Wd, bd, Wo, bo)


def kernel(x, edge_index, batch, gn0_w, gn0_b, gn0_ms, W1r, b1, W1s,
           gn1_w, gn1_b, gn1_ms, W2r, b2, W2s, gn2_w, gn2_b, gn2_ms,
           W3r, b3, W3s, Wd, bd, Wo, bo):
    br = batch.reshape(1, _N)
    pad = _NW * _K * _B - _E
    src = jnp.concatenate([edge_index[0], jnp.zeros((pad,), jnp.int32)])
    dst = jnp.concatenate([edge_index[1], jnp.full((pad,), _N, jnp.int32)])
    src3 = src.reshape(_NW, _K, _B)
    dst3 = dst.reshape(_NW, _K, _B)
    zeros = jnp.zeros((_NP, _H), _F32)

    def row(v):
        return v.reshape(1, -1)

    m1, s1 = _tc_first(x, br, row(gn0_w), row(gn0_b), row(gn0_ms),
                       W1r, row(b1), W1s)
    agg1 = _edge_agg(m1, zeros, src3, dst3)
    m2, s2 = _tc_mid(agg1, s1, br, row(gn1_w), row(gn1_b), row(gn1_ms),
                     W2r, row(b2), W2s)
    agg2 = _edge_agg(m2, zeros, src3, dst3)
    m3, s3 = _tc_mid(agg2, s2, br, row(gn2_w), row(gn2_b), row(gn2_ms),
                     W3r, row(b3), W3s)
    agg3 = _edge_agg(m3, zeros, src3, dst3)
    return _tc_final(agg3, s3, br, Wd, bd, Wo, bo)


# speedup vs baseline: 1.2457x; 1.2303x over previous
import functools

import jax
import jax.numpy as jnp
from jax import lax
from jax.experimental import pallas as pl
from jax.experimental.pallas import tpu as pltpu
from jax.experimental.pallas import tpu_sc as plsc

_N = 10000
_E = 320000
_F = 128
_H = 64
_G = 128
_C = 10

_NC = 2
_NS = 16
_NW = _NC * _NS
_B = 128
_EPW = _E // _NW
_K = 80
_NP = 10112
_RPT = _NP // _NS

_F32 = jnp.float32
_HI = lax.Precision.HIGHEST
_TC_PARAMS = pltpu.CompilerParams(vmem_limit_bytes=100 * 1024 * 1024)


def _edge_agg(m, zeros, src3, dst3):
    mesh = plsc.VectorSubcoreMesh(core_axis_name="c", subcore_axis_name="s")

    @functools.partial(
        pl.kernel,
        out_type=jax.ShapeDtypeStruct((_NC, _NP, _H), _F32),
        mesh=mesh,
        scratch_types=[
            pltpu.VMEM((_K, _B), jnp.int32),
            pltpu.VMEM((_K, _B), jnp.int32),
            pltpu.VMEM((_B, _H), _F32),
            pltpu.VMEM((_B, _H), _F32),
            pltpu.VMEM_SHARED((_NP, _H), _F32),
            pltpu.VMEM_SHARED((_NP, _H), _F32),
            pltpu.SemaphoreType.DMA,
            pltpu.SemaphoreType.DMA,
        ],
        compiler_params=pltpu.CompilerParams(use_tc_tiling_on_sc=False),
    )
    def k(m_hbm, z_hbm, src_hbm, dst_hbm, out_hbm, src_v, dst_v, rows_v,
          rows_b, m_sh, agg_sh, sem, semb):
        c = lax.axis_index("c")
        s = lax.axis_index("s")
        wid = s * _NC + c
        pltpu.sync_copy(z_hbm.at[pl.ds(s * _RPT, _RPT)],
                        agg_sh.at[pl.ds(s * _RPT, _RPT)])
        pltpu.sync_copy(m_hbm.at[pl.ds(s * _RPT, _RPT)],
                        m_sh.at[pl.ds(s * _RPT, _RPT)])
        pltpu.sync_copy(src_hbm.at[wid], src_v)
        pltpu.sync_copy(dst_hbm.at[wid], dst_v)
        plsc.subcore_barrier()

        pltpu.async_copy(m_sh.at[src_v.at[0]], rows_v, sem)

        @pl.loop(0, _K - 2, step=2)
        def _(j):
            pltpu.async_copy(m_sh.at[src_v.at[j + 1]], rows_b, semb)
            pltpu.make_async_copy(m_sh.at[src_v.at[j]], rows_v, sem).wait()
            pltpu.sync_copy(rows_v, agg_sh.at[dst_v.at[j]], add=True)
            pltpu.async_copy(m_sh.at[src_v.at[j + 2]], rows_v, sem)
            pltpu.make_async_copy(m_sh.at[src_v.at[j + 1]], rows_b,
                                  semb).wait()
            pltpu.sync_copy(rows_b, agg_sh.at[dst_v.at[j + 1]], add=True)

        pltpu.async_copy(m_sh.at[src_v.at[_K - 1]], rows_b, semb)
        pltpu.make_async_copy(m_sh.at[src_v.at[_K - 2]], rows_v, sem).wait()
        pltpu.sync_copy(rows_v, agg_sh.at[dst_v.at[_K - 2]], add=True)
        pltpu.make_async_copy(m_sh.at[src_v.at[_K - 1]], rows_b, semb).wait()
        pltpu.sync_copy(rows_b, agg_sh.at[dst_v.at[_K - 1]], add=True)

        plsc.subcore_barrier()
        pltpu.sync_copy(agg_sh.at[pl.ds(s * _RPT, _RPT)],
                        out_hbm.at[c, pl.ds(s * _RPT, _RPT)])

    return k(m, zeros, src3, dst3)


_BF = jnp.bfloat16


def _split2(v):
    hi = v.astype(_BF)
    return hi, (v - hi.astype(_F32)).astype(_BF)


def _odot(ot_bf, v):
    vh, vl = _split2(v)
    return (jnp.dot(ot_bf, vh, preferred_element_type=_F32) +
            jnp.dot(ot_bf, vl, preferred_element_type=_F32))


def _ogather(ot_bf, per_g):
    gh, gl = _split2(per_g)
    dn = (((0,), (0,)), ((), ()))
    return (lax.dot_general(ot_bf, gh, dn, preferred_element_type=_F32) +
            lax.dot_general(ot_bf, gl, dn, preferred_element_type=_F32))


def _dot3(a, b):
    ah, al = _split2(a)
    bh, bl = _split2(b)
    return (jnp.dot(ah, bh, preferred_element_type=_F32) +
            (jnp.dot(ah, bl, preferred_element_type=_F32) +
             jnp.dot(al, bh, preferred_element_type=_F32)))


def _onehot(br):
    n = br.shape[1]
    return (lax.broadcasted_iota(jnp.int32, (_G, n), 0) == br).astype(_BF)


def _norm_stats(xv, ot_bf, ms):
    cnt = jnp.maximum(
        jnp.sum(ot_bf.astype(_F32), axis=1, keepdims=True), 1.0)
    mean = _odot(ot_bf, xv) / cnt
    ex2 = _odot(ot_bf, xv * xv) / cnt
    var = ex2 - mean * mean * (ms * (2.0 - ms))
    return cnt, mean, jnp.sqrt(var + 1e-5)


def _tc_first(x, br, gw, gb, gms, Wr, b1, Ws):
    def body(x_r, br_r, gw_r, gb_r, gms_r, wr_r, b1_r, ws_r, m_o, s_o):
        ot = _onehot(br_r[...])
        _, mean, std = _norm_stats(x_r[...], ot, gms_r[...])
        meanb = _ogather(ot, mean)
        stdb = _ogather(ot, std)
        h = gw_r[...] * (x_r[...] - meanb * gms_r[...]) / stdb + gb_r[...]
        m_o[0:_N, :] = _dot3(h, wr_r[...])
        s_o[...] = _dot3(h, ws_r[...]) + b1_r[...]

    return pl.pallas_call(
        body,
        out_shape=[jax.ShapeDtypeStruct((_NP, _H), _F32),
                   jax.ShapeDtypeStruct((_N, _H), _F32)],
        compiler_params=_TC_PARAMS,
    )(x, br, gw, gb, gms, Wr, b1, Ws)


def _tc_mid(agg, s_in, br, gw, gb, gms, Wr, bn, Ws):
    def body(a_r, s_r, br_r, gw_r, gb_r, gms_r, wr_r, bn_r, ws_r, m_o, s_o):
        h = jnp.maximum(a_r[0, :_N, :] + a_r[1, :_N, :] + s_r[...], 0.0)
        ot = _onehot(br_r[...])
        _, mean, std = _norm_stats(h, ot, gms_r[...])
        meanb = _ogather(ot, mean)
        stdb = _ogather(ot, std)
        hn = gw_r[...] * (h - meanb * gms_r[...]) / stdb + gb_r[...]
        m_o[0:_N, :] = _dot3(hn, wr_r[...])
        s_o[...] = _dot3(hn, ws_r[...]) + bn_r[...]

    return pl.pallas_call(
        body,
        out_shape=[jax.ShapeDtypeStruct((_NP, _H), _F32),
                   jax.ShapeDtypeStruct((_N, _H), _F32)],
        compiler_params=_TC_PARAMS,
    )(agg, s_in, br, gw, gb, gms, Wr, bn, Ws)


def _tc_final(agg, s_in, br, Wd, bd, Wo, bo):
    def body(a_r, s_r, br_r, wd_r, bd_r, wo_r, bo_r, out_o):
        h = jnp.maximum(a_r[0, :_N, :] + a_r[1, :_N, :] + s_r[...], 0.0)
        ot = _onehot(br_r[...])
        cnt = jnp.maximum(
            jnp.sum(ot.astype(_F32), axis=1, keepdims=True), 1.0)
        g = _odot(ot, h) / cnt
        g = jnp.maximum(jnp.dot(g, wd_r[...], precision=_HI) + bd_r[...], 0.0)
        logits = jnp.dot(g, wo_r[...], precision=_HI) + bo_r[...]
        zmax = jnp.max(logits, axis=1, keepdims=True)
        ez = jnp.exp(logits - zmax)
        out_o[...] = ez / jnp.sum(ez, axis=1, keepdims=True)

    return pl.pallas_call(
        body,
        out_shape=jax.ShapeDtypeStruct((_G, _C), _F32),
        compiler_params=_TC_PARAMS,
    )(agg, s_in, br, Wd, bd, Wo, bo)


def kernel(x, edge_index, batch, gn0_w, gn0_b, gn0_ms, W1r, b1, W1s,
           gn1_w, gn1_b, gn1_ms, W2r, b2, W2s, gn2_w, gn2_b, gn2_ms,
           W3r, b3, W3s, Wd, bd, Wo, bo):
    br = batch.reshape(1, _N)
    pad = _NW * _K * _B - _E
    src = jnp.concatenate([edge_index[0], jnp.zeros((pad,), jnp.int32)])
    dst = jnp.concatenate([edge_index[1], jnp.full((pad,), _N, jnp.int32)])
    src3 = src.reshape(_NW, _K, _B)
    dst3 = dst.reshape(_NW, _K, _B)
    zeros = jnp.zeros((_NP, _H), _F32)

    def row(v):
        return v.reshape(1, -1)

    m1, s1 = _tc_first(x, br, row(gn0_w), row(gn0_b), row(gn0_ms),
                       W1r, row(b1), W1s)
    agg1 = _edge_agg(m1, zeros, src3, dst3)
    m2, s2 = _tc_mid(agg1, s1, br, row(gn1_w), row(gn1_b), row(gn1_ms),
                     W2r, row(b2), W2s)
    agg2 = _edge_agg(m2, zeros, src3, dst3)
    m3, s3 = _tc_mid(agg2, s2, br, row(gn2_w), row(gn2_b), row(gn2_ms),
                     W3r, row(b3), W3s)
    agg3 = _edge_agg(m3, zeros, src3, dst3)
    return _tc_final(agg3, s3, br, Wd, bd, Wo, bo)
